# baseline (speedup 1.0000x reference)
import functools

import jax
import jax.numpy as jnp
from jax import lax
from jax.experimental import pallas as pl
from jax.experimental.pallas import tpu as pltpu, tpu_sc as plsc

_L = 16


def _make_sc_kernel(B, S, d, NC, NS):
    NW = NC * NS
    rows_per_w = S // NW
    CH = 16
    n_chunks = rows_per_w // CH
    n_vregs = d // _L
    mesh = plsc.VectorSubcoreMesh(core_axis_name="c", subcore_axis_name="s")

    scratch = [pltpu.VMEM((CH, d), jnp.float32)] * (2 + 2 * B)
    scratch += [pltpu.SemaphoreType.DMA] * 6

    @functools.partial(
        pl.kernel,
        out_type=jax.ShapeDtypeStruct((B, S, d), jnp.float32),
        mesh=mesh,
        scratch_types=scratch,
    )
    def sc_kernel(x_hbm, tbl_hbm, out_hbm, *refs):
        tbl_v = refs[0:2]
        grp = (refs[2:2 + B], refs[2 + B:2 + 2 * B])
        st = refs[2 + 2 * B:4 + 2 * B]
        sx = refs[4 + 2 * B:6 + 2 * B]
        so = refs[6 + 2 * B:8 + 2 * B]
        wid = lax.axis_index("s") * NC + lax.axis_index("c")
        base = wid * rows_per_w

        def rows(c):
            return pl.ds(base + c * CH, CH)

        def start_tbl(c):
            return pltpu.async_copy(tbl_hbm.at[rows(c)], tbl_v[c % 2],
                                    st[c % 2])

        def start_x_quad(c):
            g = c % 2
            return [pltpu.async_copy(x_hbm.at[b, rows(c)], grp[g][b], sx[g])
                    for b in range(B)]

        def start_out_quad(c):
            g = c % 2
            return [pltpu.async_copy(grp[g][b], out_hbm.at[b, rows(c)], so[g])
                    for b in range(B)]

        tbl_cp = {0: start_tbl(0)}
        x_cp = {0: start_x_quad(0)}
        if n_chunks > 1:
            x_cp[1] = start_x_quad(1)
        out_cp = {}

        for c in range(n_chunks):
            if c + 1 < n_chunks:
                tbl_cp[c + 1] = start_tbl(c + 1)
            tbl_cp.pop(c).wait()
            for cp in x_cp.pop(c):
                cp.wait()
            tbl = tbl_v[c % 2]
            bufs = grp[c % 2]

            @plsc.parallel_loop(0, CH, 1)
            def _(r):
                @plsc.parallel_loop(0, n_vregs, 1, unroll=8)
                def _(j):
                    sl = pl.ds(j * _L, _L)
                    t = tbl[r, sl]
                    for b in range(B):
                        bufs[b][r, sl] = bufs[b][r, sl] + t

            out_cp[c] = start_out_quad(c)
            if c + 2 < n_chunks:
                g = c % 2
                nxt = []
                for b, cp in enumerate(out_cp.pop(c)):
                    cp.wait()
                    nxt.append(pltpu.async_copy(
                        x_hbm.at[b, rows(c + 2)], grp[g][b], sx[g]))
                x_cp[c + 2] = nxt

        for cps in out_cp.values():
            for cp in cps:
                cp.wait()

    return sc_kernel


def kernel(x, pos_embedding):
    B, D1, D2, d = x.shape
    S = D1 * D2
    info = plsc.get_sparse_core_info()
    sc = _make_sc_kernel(B, S, d, info.num_cores, info.num_subcores)
    out = sc(x.reshape(B, S, d), pos_embedding)
    return out.reshape(B, D1, D2, d)


# speedup vs baseline: 1.0322x; 1.0322x over previous
import functools

import jax
import jax.numpy as jnp
from jax import lax
from jax.experimental import pallas as pl
from jax.experimental.pallas import tpu as pltpu, tpu_sc as plsc

_L = 16


def _make_sc_kernel(B, S, d, NC, NS):
    NW = NC * NS
    rows_per_w = S // NW
    CH = 16
    n_chunks = rows_per_w // CH
    n_vregs = d // _L
    mesh = plsc.VectorSubcoreMesh(core_axis_name="c", subcore_axis_name="s")

    scratch = [pltpu.VMEM((CH, d), jnp.float32)] * (2 + 2 * B)
    scratch += [pltpu.SemaphoreType.DMA] * 6

    @functools.partial(
        pl.kernel,
        out_type=jax.ShapeDtypeStruct((B, S, d), jnp.float32),
        mesh=mesh,
        scratch_types=scratch,
    )
    def sc_kernel(x_hbm, tbl_hbm, out_hbm, *refs):
        tbl_v = refs[0:2]
        grp = (refs[2:2 + B], refs[2 + B:2 + 2 * B])
        st = refs[2 + 2 * B:4 + 2 * B]
        sx = refs[4 + 2 * B:6 + 2 * B]
        so = refs[6 + 2 * B:8 + 2 * B]
        wid = lax.axis_index("s") * NC + lax.axis_index("c")
        base = wid * rows_per_w

        def rows(c):
            return pl.ds(base + c * CH, CH)

        def start_tbl(c):
            return pltpu.async_copy(tbl_hbm.at[rows(c)], tbl_v[c % 2],
                                    st[c % 2])

        def start_x_quad(c):
            g = c % 2
            return [pltpu.async_copy(x_hbm.at[b, rows(c)], grp[g][b], sx[g])
                    for b in range(B)]

        def start_out_quad(c):
            g = c % 2
            return [pltpu.async_copy(grp[g][b], out_hbm.at[b, rows(c)], so[g])
                    for b in range(B)]

        tbl_cp = {0: start_tbl(0)}
        x_cp = {0: start_x_quad(0)}
        if n_chunks > 1:
            x_cp[1] = start_x_quad(1)
        out_cp = {}

        for c in range(n_chunks):
            if c + 1 < n_chunks:
                tbl_cp[c + 1] = start_tbl(c + 1)
            tbl_cp.pop(c).wait()
            for cp in x_cp.pop(c):
                cp.wait()
            tbl = tbl_v[c % 2]
            bufs = grp[c % 2]

            @plsc.parallel_loop(0, CH, 1)
            def _(r):
                @plsc.parallel_loop(0, n_vregs, 1, unroll=4)
                def _(j):
                    sl = pl.ds(j * _L, _L)
                    t = tbl[r, sl]
                    for b in range(B):
                        bufs[b][r, sl] = bufs[b][r, sl] + t

            out_cp[c] = start_out_quad(c)
            if c + 2 < n_chunks:
                g = c % 2
                nxt = []
                for b, cp in enumerate(out_cp.pop(c)):
                    cp.wait()
                    nxt.append(pltpu.async_copy(
                        x_hbm.at[b, rows(c + 2)], grp[g][b], sx[g]))
                x_cp[c + 2] = nxt

        for cps in out_cp.values():
            for cp in cps:
                cp.wait()

    return sc_kernel


def kernel(x, pos_embedding):
    B, D1, D2, d = x.shape
    S = D1 * D2
    info = plsc.get_sparse_core_info()
    sc = _make_sc_kernel(B, S, d, info.num_cores, info.num_subcores)
    out = sc(x.reshape(B, S, d), pos_embedding)
    return out.reshape(B, D1, D2, d)
